# baseline (speedup 1.0000x reference)
import functools

import jax
import jax.numpy as jnp
from jax import lax
from jax.experimental import pallas as pl
from jax.experimental.pallas import tpu as pltpu
import jax.experimental.pallas.tpu_sc as plsc

N = 32768
B = 64
NG = 512
E = 524288
D = 128
TOT = 97
K = 30
C1, C2 = 16, 32
KW2 = 5
PADC = 128

TPB = 512
CH = 128
EPAD = 24576

_INTERP = False


def _sc_segsum(table, src_s, dst_s, cuts, zeros):
    mesh = plsc.VectorSubcoreMesh(core_axis_name="c", subcore_axis_name="s")

    @functools.partial(
        pl.kernel,
        out_type=jax.ShapeDtypeStruct((N, CH), jnp.float32),
        mesh=mesh,
        scratch_types=[
            pltpu.VMEM((96,), jnp.int32),
            pltpu.VMEM((128,), jnp.int32),
            pltpu.VMEM((128,), jnp.int32),
            pltpu.VMEM((128,), jnp.int32),
            pltpu.VMEM((128, CH), jnp.float32),
            pltpu.VMEM_SHARED((16 * (TPB + 8), CH), jnp.float32),
            pltpu.SemaphoreType.DMA,
        ],
    )
    def k(table_h, src_h, dst_h, cuts_h, zeros_h, out_h,
          cutsv, dstb, srcb, lbuf, rowb, acc, sem):
        cc = lax.axis_index("c")
        ss = lax.axis_index("s")
        w = ss * 2 + cc
        rbase = ss * (TPB + 8)
        pltpu.sync_copy(cuts_h, cutsv)
        for half in range(2):
            widx = half * 32 + w
            base = widx * TPB
            pltpu.sync_copy(zeros_h, acc.at[pl.ds(rbase, TPB + 8)])
            cv = cutsv[pl.ds(widx, 16)]
            lo = cv[0]
            hi = cv[1]
            a0 = (lo // 8) * 8
            trips = (hi - a0 + 127) // 128

            def body(i, carry):
                off = pl.multiple_of(a0 + i * 128, 8)
                pltpu.sync_copy(dst_h.at[pl.ds(off, 128)], dstb)
                pltpu.sync_copy(src_h.at[pl.ds(off, 128)], srcb)
                for g in range(8):
                    d16 = dstb[pl.ds(g * 16, 16)]
                    m = (d16 >= base) & (d16 < base + TPB)
                    lbuf[pl.ds(g * 16, 16)] = rbase + jnp.where(
                        m, d16 - base, TPB)
                pltpu.async_copy(table_h.at[srcb], rowb, sem).wait()
                pltpu.sync_copy(rowb, acc.at[lbuf], add=True)
                return carry

            lax.fori_loop(0, trips, body, 0)
            pltpu.sync_copy(acc.at[pl.ds(rbase, TPB)],
                            out_h.at[pl.ds(base, TPB)])

    return k(table, src_s, dst_s, cuts, zeros)


_RB = 2048


def _layer_body(seg_ref, h_ref, dp_ref, b_ref, w_ref, o_ref):
    pooled = seg_ref[...] + h_ref[...]
    lin = jnp.dot(pooled, w_ref[...], preferred_element_type=jnp.float32) \
        + b_ref[...]
    o_ref[...] = jnp.tanh(lin / dp_ref[...])


def _layer(seg, h, dp, b, Wn):
    din = h.shape[1]
    wout = Wn.shape[1]
    return pl.pallas_call(
        _layer_body,
        grid=(N // _RB,),
        in_specs=[
            pl.BlockSpec((_RB, din), lambda i: (i, 0)),
            pl.BlockSpec((_RB, din), lambda i: (i, 0)),
            pl.BlockSpec((_RB, 1), lambda i: (i, 0)),
            pl.BlockSpec((1, wout), lambda i: (0, 0)),
            pl.BlockSpec((din, wout), lambda i: (0, 0)),
        ],
        out_specs=pl.BlockSpec((_RB, wout), lambda i: (i, 0)),
        out_shape=jax.ShapeDtypeStruct((N, wout), jnp.float32),
        interpret=_INTERP,
    )(seg, h, dp, b, Wn)


def _deg_body(s0_ref, s1_ref, dp_ref):
    dp_ref[...] = (s1_ref[...] - s0_ref[...]).astype(jnp.float32) + 1.0


def _deg(starts0, starts1):
    return pl.pallas_call(
        _deg_body,
        grid=(N // _RB,),
        in_specs=[
            pl.BlockSpec((_RB, 1), lambda i: (i, 0)),
            pl.BlockSpec((_RB, 1), lambda i: (i, 0)),
        ],
        out_specs=pl.BlockSpec((_RB, 1), lambda i: (i, 0)),
        out_shape=jax.ShapeDtypeStruct((N, 1), jnp.float32),
        interpret=_INTERP,
    )(starts0, starts1)


def _topk_body(s_ref, sel_ref):
    s = s_ref[...]
    ii = lax.broadcasted_iota(jnp.int32, (B, NG), 1)
    cols = []
    for _ in range(K):
        m = jnp.max(s, axis=1, keepdims=True)
        idxv = jnp.where(s == m, ii, NG)
        selk = jnp.min(idxv, axis=1, keepdims=True)
        cols.append(selk)
        s = jnp.where(ii == selk, -jnp.inf, s)
    sel_ref[...] = jnp.concatenate(cols, axis=1)


def _topk(scores):
    return pl.pallas_call(
        _topk_body,
        grid=(1,),
        in_specs=[pl.BlockSpec((B, NG), lambda i: (0, 0))],
        out_specs=pl.BlockSpec((B, K), lambda i: (0, 0)),
        out_shape=jax.ShapeDtypeStruct((B, K), jnp.int32),
        interpret=_INTERP,
    )(scores)


def _head_body(sel_ref, h1_ref, h2_ref, h3_ref, h4_ref, w1_ref, b1c_ref,
               se_ref, w2_ref, b2c_ref, ow_ref, ob_ref, out_ref, cats_ref):
    cats_ref[...] = jnp.concatenate(
        [h1_ref[...], h2_ref[...], h3_ref[...], h4_ref[...],
         jnp.zeros((NG, PADC - TOT), jnp.float32)], axis=1)
    pooled = jnp.concatenate(
        [cats_ref[pl.ds(sel_ref[0, 0, k], 1), :] for k in range(K)], axis=0)
    y1 = jax.nn.relu(
        jnp.dot(pooled, w1_ref[...], preferred_element_type=jnp.float32)
        + b1c_ref[...])
    ya = jnp.dot(se_ref[0], y1, preferred_element_type=jnp.float32)
    yb = jnp.dot(se_ref[1], y1, preferred_element_type=jnp.float32)
    yp = jnp.maximum(ya, yb)
    acc2 = jnp.zeros((11, C2), jnp.float32)
    for dk in range(KW2):
        acc2 = acc2 + jnp.dot(yp[dk:dk + 11, :], w2_ref[dk],
                              preferred_element_type=jnp.float32)
    y2 = jax.nn.relu(acc2 + b2c_ref[...])
    acc = jnp.zeros((1, 128), jnp.float32) + ob_ref[...]
    for j in range(11):
        acc = acc + jnp.dot(y2[j:j + 1, :], ow_ref[j],
                            preferred_element_type=jnp.float32)
    out_ref[...] = jnp.expand_dims(jax.nn.relu(acc), 0)


def _head(sel3, h1, h2, h3, h4c, w1t, b1c, se, w2r, b2c, owr, obc):
    return pl.pallas_call(
        _head_body,
        grid=(B,),
        in_specs=[
            pl.BlockSpec((1, 1, K), lambda b: (b, 0, 0),
                         memory_space=pltpu.SMEM),
            pl.BlockSpec((NG, 32), lambda b: (b, 0)),
            pl.BlockSpec((NG, 32), lambda b: (b, 0)),
            pl.BlockSpec((NG, 32), lambda b: (b, 0)),
            pl.BlockSpec((NG, 1), lambda b: (b, 0)),
            pl.BlockSpec((PADC, C1), lambda b: (0, 0)),
            pl.BlockSpec((1, C1), lambda b: (0, 0)),
            pl.BlockSpec((2, 15, K), lambda b: (0, 0, 0)),
            pl.BlockSpec((KW2, C1, C2), lambda b: (0, 0, 0)),
            pl.BlockSpec((1, C2), lambda b: (0, 0)),
            pl.BlockSpec((11, C2, 128), lambda b: (0, 0, 0)),
            pl.BlockSpec((1, 128), lambda b: (0, 0)),
        ],
        out_specs=pl.BlockSpec((1, 1, 128), lambda b: (b, 0, 0)),
        out_shape=jax.ShapeDtypeStruct((B, 1, 128), jnp.float32),
        scratch_shapes=[pltpu.VMEM((NG, PADC), jnp.float32)],
        interpret=_INTERP,
    )(sel3, h1, h2, h3, h4c, w1t, b1c, se, w2r, b2c, owr, obc)


def kernel(node_feat, edge_index, W0, b0, W1, b1, W2, b2, W3, b3,
           conv1_w, conv1_b, conv2_w, conv2_b, out_w, out_b):
    src = edge_index[0]
    dst = edge_index[1]

    perm = jnp.argsort(dst, stable=True)
    src_s = jnp.concatenate([src[perm], jnp.zeros((EPAD,), jnp.int32)])
    dst_s = jnp.concatenate([dst[perm], jnp.full((EPAD,), N, jnp.int32)])
    starts = jnp.searchsorted(dst_s[:E], jnp.arange(N + 1, dtype=jnp.int32)
                              ).astype(jnp.int32)
    cuts = jnp.concatenate([starts[::TPB], jnp.full((31,), E, jnp.int32)])
    zeros = jnp.zeros((TPB + 8, CH), jnp.float32)
    zpad = jnp.zeros((N, CH - 32), jnp.float32)
    starts0 = starts[:N].reshape(N, 1)
    starts1 = starts[1:].reshape(N, 1)

    b0r = b0.reshape(1, 32)
    b1r = b1.reshape(1, 32)
    b2r = b2.reshape(1, 32)
    b3r = b3.reshape(1, 1)
    w1t = jnp.zeros((PADC, C1), jnp.float32).at[:TOT].set(conv1_w[:, 0, :].T)
    b1c = conv1_b.reshape(1, C1)
    eye30 = jnp.eye(K, dtype=jnp.float32)
    se = jnp.stack([eye30[0::2], eye30[1::2]])
    w2r = conv2_w.transpose(2, 1, 0)
    b2c = conv2_b.reshape(1, C2)
    owr = out_w.reshape(C2, 11, 128).transpose(1, 0, 2)
    obc = out_b.reshape(1, 128)

    dp = _deg(starts0, starts1)

    seg0 = _sc_segsum(node_feat, src_s, dst_s, cuts, zeros)
    h1 = _layer(seg0, node_feat, dp, b0r, W0)
    seg1 = _sc_segsum(jnp.concatenate([h1, zpad], axis=1),
                      src_s, dst_s, cuts, zeros)[:, :32]
    h2 = _layer(seg1, h1, dp, b1r, W1)
    seg2 = _sc_segsum(jnp.concatenate([h2, zpad], axis=1),
                      src_s, dst_s, cuts, zeros)[:, :32]
    h3 = _layer(seg2, h2, dp, b2r, W2)
    seg3 = _sc_segsum(jnp.concatenate([h3, zpad], axis=1),
                      src_s, dst_s, cuts, zeros)[:, :32]
    h4 = _layer(seg3, h3, dp, b3r, W3)

    sel = _topk(h4.reshape(B, NG))
    sel3 = sel.reshape(B, 1, K)

    out3 = _head(sel3, h1, h2, h3, h4, w1t, b1c, se, w2r, b2c, owr, obc)
    return out3.reshape(B, 128)
